# baseline (speedup 1.0000x reference)
import functools

import jax
import jax.numpy as jnp
from jax import lax
from jax.experimental import pallas as pl
from jax.experimental.pallas import tpu as pltpu
from jax.experimental.pallas import tpu_sc as plsc


def _make_gather(V, D, B, NW, chunk):
    assert B % (NW * chunk) == 0
    b_per_w = B // NW
    n_chunks = b_per_w // chunk
    mesh = plsc.VectorSubcoreMesh(core_axis_name="c", subcore_axis_name="s")

    @functools.partial(
        pl.kernel,
        mesh=mesh,
        compiler_params=pltpu.CompilerParams(use_tc_tiling_on_sc=False),
        out_type=jax.ShapeDtypeStruct((B, D), jnp.float32),
        scratch_types=[
            pltpu.VMEM((chunk,), jnp.int32),
            pltpu.VMEM((chunk, D), jnp.float32),
            pltpu.SemaphoreType.DMA,
        ],
    )
    def gather_kernel(idx_hbm, table_hbm, out_hbm, idx_v, rows_v, sem):
        nc = lax.axis_size("c")
        wid = lax.axis_index("s") * nc + lax.axis_index("c")
        base = wid * b_per_w

        def body(g, carry):
            off = base + g * chunk
            pltpu.sync_copy(idx_hbm.at[pl.ds(off, chunk)], idx_v)
            pltpu.async_copy(table_hbm.at[idx_v], rows_v, sem).wait()
            pltpu.sync_copy(rows_v, out_hbm.at[pl.ds(off, chunk)])
            return carry

        lax.fori_loop(0, n_chunks, body, 0)

    return gather_kernel


def kernel(x, table):
    V, D = table.shape
    orig_shape = x.shape
    idx = jnp.reshape(x, (-1,)).astype(jnp.int32)
    B = idx.shape[0]
    info = plsc.get_sparse_core_info()
    NW = info.num_cores * info.num_subcores
    chunk = 800
    out = _make_gather(V, D, B, NW, chunk)(idx, table)
    return jnp.reshape(out, orig_shape + (D,))


# speedup vs baseline: 1.0210x; 1.0210x over previous
import functools

import jax
import jax.numpy as jnp
from jax import lax
from jax.experimental import pallas as pl
from jax.experimental.pallas import tpu as pltpu
from jax.experimental.pallas import tpu_sc as plsc


def _make_gather(V, D, B, NW, chunk):
    assert B % (NW * chunk) == 0
    b_per_w = B // NW
    n_chunks = b_per_w // chunk
    assert n_chunks >= 2 and n_chunks % 2 == 0
    mesh = plsc.VectorSubcoreMesh(core_axis_name="c", subcore_axis_name="s")

    @functools.partial(
        pl.kernel,
        mesh=mesh,
        compiler_params=pltpu.CompilerParams(use_tc_tiling_on_sc=False),
        out_type=jax.ShapeDtypeStruct((B, D), jnp.float32),
        scratch_types=[
            pltpu.VMEM((b_per_w,), jnp.int32),
            pltpu.VMEM((chunk, D), jnp.float32),
            pltpu.VMEM((chunk, D), jnp.float32),
            pltpu.SemaphoreType.DMA,
            pltpu.SemaphoreType.DMA,
            pltpu.SemaphoreType.DMA,
            pltpu.SemaphoreType.DMA,
        ],
    )
    def gather_kernel(idx_hbm, table_hbm, out_hbm, idx_v, rows0, rows1,
                      gsem0, gsem1, wsem0, wsem1):
        nc = lax.axis_size("c")
        wid = lax.axis_index("s") * nc + lax.axis_index("c")
        base = wid * b_per_w

        rows = (rows0, rows1)
        gsem = (gsem0, gsem1)
        wsem = (wsem0, wsem1)

        pltpu.sync_copy(idx_hbm.at[pl.ds(base, b_per_w)], idx_v)

        def start_gather(t):
            b = t % 2
            pltpu.async_copy(
                table_hbm.at[idx_v.at[pl.ds(t * chunk, chunk)]], rows[b],
                gsem[b])

        def wait_gather(t):
            b = t % 2
            pltpu.make_async_copy(
                table_hbm.at[idx_v.at[pl.ds(t * chunk, chunk)]], rows[b],
                gsem[b]).wait()

        def start_wb(t):
            b = t % 2
            pltpu.async_copy(rows[b], out_hbm.at[pl.ds(base + t * chunk, chunk)],
                             wsem[b])

        def wait_wb(t):
            b = t % 2
            pltpu.make_async_copy(
                rows[b], out_hbm.at[pl.ds(base + t * chunk, chunk)],
                wsem[b]).wait()

        start_gather(0)
        start_gather(1)
        wait_gather(0)
        start_wb(0)

        def body(k, carry):
            for b in range(2):
                t = 2 * k + 1 + b
                cur = (1 + b) % 2
                oth = (cur + 1) % 2
                pltpu.make_async_copy(
                    rows[oth],
                    out_hbm.at[pl.ds(base + (t - 1) * chunk, chunk)],
                    wsem[oth]).wait()
                pltpu.async_copy(
                    table_hbm.at[idx_v.at[pl.ds((t + 1) * chunk, chunk)]],
                    rows[oth], gsem[oth])
                pltpu.make_async_copy(
                    table_hbm.at[idx_v.at[pl.ds(t * chunk, chunk)]],
                    rows[cur], gsem[cur]).wait()
                pltpu.async_copy(
                    rows[cur],
                    out_hbm.at[pl.ds(base + t * chunk, chunk)], wsem[cur])
            return carry

        lax.fori_loop(0, (n_chunks - 2) // 2, body, 0)

        t_last = n_chunks - 1
        wait_wb(t_last - 1)
        wait_gather(t_last)
        start_wb(t_last)
        wait_wb(t_last)

    return gather_kernel


def kernel(x, table):
    V, D = table.shape
    orig_shape = x.shape
    idx = jnp.reshape(x, (-1,)).astype(jnp.int32)
    B = idx.shape[0]
    info = plsc.get_sparse_core_info()
    NW = info.num_cores * info.num_subcores
    chunk = 800
    out = _make_gather(V, D, B, NW, chunk)(idx, table)
    return jnp.reshape(out, orig_shape + (D,))


# speedup vs baseline: 1.0521x; 1.0304x over previous
import functools

import jax
import jax.numpy as jnp
from jax import lax
from jax.experimental import pallas as pl
from jax.experimental.pallas import tpu as pltpu
from jax.experimental.pallas import tpu_sc as plsc


def _make_gather(V, D, B, NW, chunk):
    assert B % (NW * chunk) == 0
    b_per_w = B // NW
    n_chunks = b_per_w // chunk
    assert n_chunks >= 2 and n_chunks % 2 == 0
    mesh = plsc.VectorSubcoreMesh(core_axis_name="c", subcore_axis_name="s")

    @functools.partial(
        pl.kernel,
        mesh=mesh,
        compiler_params=pltpu.CompilerParams(use_tc_tiling_on_sc=False),
        out_type=jax.ShapeDtypeStruct((B, D), jnp.float32),
        scratch_types=[
            pltpu.VMEM((b_per_w,), jnp.int32),
            pltpu.VMEM((chunk, D), jnp.float32),
            pltpu.VMEM((chunk, D), jnp.float32),
            pltpu.SemaphoreType.DMA,
            pltpu.SemaphoreType.DMA,
            pltpu.SemaphoreType.DMA,
            pltpu.SemaphoreType.DMA,
        ],
    )
    def gather_kernel(idx_hbm, table_hbm, out_hbm, idx_v, rows0, rows1,
                      gsem0, gsem1, wsem0, wsem1):
        nc = lax.axis_size("c")
        wid = lax.axis_index("s") * nc + lax.axis_index("c")
        base = wid * b_per_w

        rows = (rows0, rows1)
        gsem = (gsem0, gsem1)
        wsem = (wsem0, wsem1)

        pltpu.sync_copy(idx_hbm.at[pl.ds(base, b_per_w)], idx_v)

        def start_gather(t):
            b = t % 2
            pltpu.async_copy(
                table_hbm.at[idx_v.at[pl.ds(t * chunk, chunk)]], rows[b],
                gsem[b])

        def wait_gather(t):
            b = t % 2
            pltpu.make_async_copy(
                table_hbm.at[idx_v.at[pl.ds(t * chunk, chunk)]], rows[b],
                gsem[b]).wait()

        def start_wb(t):
            b = t % 2
            pltpu.async_copy(rows[b], out_hbm.at[pl.ds(base + t * chunk, chunk)],
                             wsem[b])

        def wait_wb(t):
            b = t % 2
            pltpu.make_async_copy(
                rows[b], out_hbm.at[pl.ds(base + t * chunk, chunk)],
                wsem[b]).wait()

        start_gather(0)
        start_gather(1)
        wait_gather(0)
        start_wb(0)

        def body(k, carry):
            for b in range(2):
                t = 2 * k + 1 + b
                cur = (1 + b) % 2
                oth = (cur + 1) % 2
                pltpu.make_async_copy(
                    rows[oth],
                    out_hbm.at[pl.ds(base + (t - 1) * chunk, chunk)],
                    wsem[oth]).wait()
                pltpu.async_copy(
                    table_hbm.at[idx_v.at[pl.ds((t + 1) * chunk, chunk)]],
                    rows[oth], gsem[oth])
                pltpu.make_async_copy(
                    table_hbm.at[idx_v.at[pl.ds(t * chunk, chunk)]],
                    rows[cur], gsem[cur]).wait()
                pltpu.async_copy(
                    rows[cur],
                    out_hbm.at[pl.ds(base + t * chunk, chunk)], wsem[cur])
            return carry

        lax.fori_loop(0, (n_chunks - 2) // 2, body, 0)

        t_last = n_chunks - 1
        wait_wb(t_last - 1)
        wait_gather(t_last)
        start_wb(t_last)
        wait_wb(t_last)

    return gather_kernel


def kernel(x, table):
    V, D = table.shape
    B0, B1 = x.shape
    B = B0 * B1
    idx = jnp.reshape(jnp.transpose(x), (-1,)).astype(jnp.int32)
    info = plsc.get_sparse_core_info()
    NW = info.num_cores * info.num_subcores
    chunk = 800
    out = _make_gather(V, D, B, NW, chunk)(idx, table)
    return jnp.transpose(jnp.reshape(out, (B1, B0, D)), (1, 0, 2))


# speedup vs baseline: 1.1458x; 1.0891x over previous
import functools

import jax
import jax.numpy as jnp
from jax import lax
from jax.experimental import pallas as pl
from jax.experimental.pallas import tpu as pltpu
from jax.experimental.pallas import tpu_sc as plsc


def _make_gather(V, D, B0, B1, NJ, NP, chunk_p):
    P = B0 // 2
    p_per_w = P // NP
    n_chunks = p_per_w // chunk_p
    assert B1 % NJ == 0 and P % NP == 0 and p_per_w % chunk_p == 0
    j_per_w = B1 // NJ
    mesh = plsc.VectorSubcoreMesh(core_axis_name="c", subcore_axis_name="s")

    @functools.partial(
        pl.kernel,
        mesh=mesh,
        compiler_params=pltpu.CompilerParams(use_tc_tiling_on_sc=False),
        out_type=jax.ShapeDtypeStruct((B1, P, 2 * D), jnp.float32),
        scratch_types=[
            pltpu.VMEM((j_per_w, 2, p_per_w), jnp.int32),
            pltpu.VMEM((chunk_p, D), jnp.float32),
            pltpu.VMEM((chunk_p, D), jnp.float32),
            pltpu.VMEM((chunk_p, D), jnp.float32),
            pltpu.VMEM((chunk_p, D), jnp.float32),
            pltpu.SemaphoreType.DMA,
            pltpu.SemaphoreType.DMA,
            pltpu.SemaphoreType.DMA,
            pltpu.SemaphoreType.DMA,
        ],
    )
    def gather_kernel(idx_hbm, table_hbm, out_hbm, idx_v, bufe0, bufo0,
                      bufe1, bufo1, gsem0, gsem1, wsem0, wsem1):
        nc = lax.axis_size("c")
        wid = lax.axis_index("s") * nc + lax.axis_index("c")
        a = wid // NP
        b = wid % NP
        j0 = a * j_per_w
        pbase = b * p_per_w

        buf = ((bufe0, bufo0), (bufe1, bufo1))
        gsem = (gsem0, gsem1)
        wsem = (wsem0, wsem1)

        pltpu.sync_copy(
            idx_hbm.at[pl.ds(j0, j_per_w), :, pl.ds(pbase, p_per_w)], idx_v)

        def idx_slice(t, par):
            jj = t // n_chunks
            poff = (t % n_chunks) * chunk_p
            return idx_v.at[jj, par, pl.ds(poff, chunk_p)]

        def start_gather(t, s):
            pltpu.async_copy(
                table_hbm.at[idx_slice(t, 0)], buf[s][0], gsem[s])
            pltpu.async_copy(
                table_hbm.at[idx_slice(t, 1)], buf[s][1], gsem[s])

        def wait_gather(t, s):
            pltpu.make_async_copy(
                table_hbm.at[idx_slice(t, 0)], buf[s][0], gsem[s]).wait()
            pltpu.make_async_copy(
                table_hbm.at[idx_slice(t, 1)], buf[s][1], gsem[s]).wait()

        def out_slice(t, par):
            jj = t // n_chunks
            poff = pbase + (t % n_chunks) * chunk_p
            return out_hbm.at[j0 + jj, pl.ds(poff, chunk_p),
                              pl.ds(par * D, D)]

        def start_wb(t, s):
            pltpu.async_copy(buf[s][0], out_slice(t, 0), wsem[s])
            pltpu.async_copy(buf[s][1], out_slice(t, 1), wsem[s])

        def wait_wb(t, s):
            pltpu.make_async_copy(buf[s][0], out_slice(t, 0), wsem[s]).wait()
            pltpu.make_async_copy(buf[s][1], out_slice(t, 1), wsem[s]).wait()

        n_tot = j_per_w * n_chunks
        assert n_tot >= 2 and n_tot % 2 == 0

        start_gather(0, 0)
        start_gather(1, 1)
        wait_gather(0, 0)
        start_wb(0, 0)

        def body(k, carry):
            for s0 in range(2):
                t = 2 * k + 1 + s0
                cur = (1 + s0) % 2
                oth = (cur + 1) % 2
                wait_wb(t - 1, oth)
                start_gather(t + 1, oth)
                wait_gather(t, cur)
                start_wb(t, cur)
            return carry

        lax.fori_loop(0, (n_tot - 2) // 2, body, 0)

        t_last = n_tot - 1
        wait_wb(t_last - 1, 0)
        wait_gather(t_last, 1)
        start_wb(t_last, 1)
        wait_wb(t_last, 1)

    return gather_kernel


def kernel(x, table):
    V, D = table.shape
    B0, B1 = x.shape
    idx = jnp.transpose(
        jnp.reshape(jnp.transpose(x), (B1, B0 // 2, 2)), (0, 2, 1)
    ).astype(jnp.int32)
    NJ, NP = 8, 4
    chunk_p = 256
    out = _make_gather(V, D, B0, B1, NJ, NP, chunk_p)(idx, table)
    out4 = jnp.reshape(out, (B1, B0 // 2, 2, D))
    return jnp.reshape(jnp.transpose(out4, (1, 2, 0, 3)), (B0, B1, D))


# speedup vs baseline: 1.1497x; 1.0034x over previous
import functools

import jax
import jax.numpy as jnp
from jax import lax
from jax.experimental import pallas as pl
from jax.experimental.pallas import tpu as pltpu
from jax.experimental.pallas import tpu_sc as plsc


def _make_gather(V, D, B0, B1, NJ, NP, chunk_p):
    P = B0 // 2
    p_per_w = P // NP
    n_chunks = p_per_w // chunk_p
    assert B1 % NJ == 0 and P % NP == 0 and p_per_w % chunk_p == 0
    j_per_w = B1 // NJ
    mesh = plsc.VectorSubcoreMesh(core_axis_name="c", subcore_axis_name="s")

    @functools.partial(
        pl.kernel,
        mesh=mesh,
        compiler_params=pltpu.CompilerParams(use_tc_tiling_on_sc=False),
        out_type=jax.ShapeDtypeStruct((B1, P, 2 * D), jnp.float32),
        scratch_types=[
            pltpu.VMEM((j_per_w, 2, p_per_w), jnp.int32),
            pltpu.VMEM((chunk_p, D), jnp.float32),
            pltpu.VMEM((chunk_p, D), jnp.float32),
            pltpu.VMEM((chunk_p, D), jnp.float32),
            pltpu.VMEM((chunk_p, D), jnp.float32),
            pltpu.VMEM((chunk_p, D), jnp.float32),
            pltpu.VMEM((chunk_p, D), jnp.float32),
            pltpu.SemaphoreType.DMA,
            pltpu.SemaphoreType.DMA,
            pltpu.SemaphoreType.DMA,
            pltpu.SemaphoreType.DMA,
            pltpu.SemaphoreType.DMA,
            pltpu.SemaphoreType.DMA,
        ],
    )
    def gather_kernel(idx_hbm, table_hbm, out_hbm, idx_v, bufe0, bufo0,
                      bufe1, bufo1, bufe2, bufo2,
                      gsem0, gsem1, gsem2, wsem0, wsem1, wsem2):
        nc = lax.axis_size("c")
        wid = lax.axis_index("s") * nc + lax.axis_index("c")
        a = wid // NP
        b = wid % NP
        j0 = a * j_per_w
        pbase = b * p_per_w

        buf = ((bufe0, bufo0), (bufe1, bufo1), (bufe2, bufo2))
        gsem = (gsem0, gsem1, gsem2)
        wsem = (wsem0, wsem1, wsem2)

        pltpu.sync_copy(
            idx_hbm.at[pl.ds(j0, j_per_w), :, pl.ds(pbase, p_per_w)], idx_v)

        def idx_slice(t, par):
            jj = t // n_chunks
            poff = (t % n_chunks) * chunk_p
            return idx_v.at[jj, par, pl.ds(poff, chunk_p)]

        def start_gather(t, s):
            pltpu.async_copy(
                table_hbm.at[idx_slice(t, 0)], buf[s][0], gsem[s])
            pltpu.async_copy(
                table_hbm.at[idx_slice(t, 1)], buf[s][1], gsem[s])

        def wait_gather(t, s):
            pltpu.make_async_copy(
                table_hbm.at[idx_slice(t, 0)], buf[s][0], gsem[s]).wait()
            pltpu.make_async_copy(
                table_hbm.at[idx_slice(t, 1)], buf[s][1], gsem[s]).wait()

        def out_slice(t, par):
            jj = t // n_chunks
            poff = pbase + (t % n_chunks) * chunk_p
            return out_hbm.at[j0 + jj, pl.ds(poff, chunk_p),
                              pl.ds(par * D, D)]

        def start_wb(t, s):
            pltpu.async_copy(buf[s][0], out_slice(t, 0), wsem[s])
            pltpu.async_copy(buf[s][1], out_slice(t, 1), wsem[s])

        def wait_wb(t, s):
            pltpu.make_async_copy(buf[s][0], out_slice(t, 0), wsem[s]).wait()
            pltpu.make_async_copy(buf[s][1], out_slice(t, 1), wsem[s]).wait()

        n_tot = j_per_w * n_chunks
        nb = 3
        assert n_tot >= nb + 1

        def step(t, with_gather=True):
            s = t % nb if isinstance(t, int) else None
            assert s is not None
            if t >= 1:
                wait_wb(t - 1, (t - 1) % nb)
            if with_gather and t + nb - 1 <= n_tot - 1:
                start_gather(t + nb - 1, (t + nb - 1) % nb)
            wait_gather(t, s)
            start_wb(t, s)

        for t in range(nb - 1):
            start_gather(t, t)
        step(0)

        n_steady = n_tot - 1
        n_blocks = n_steady // nb
        rem = n_steady % nb

        def body(k, carry):
            for s0 in range(nb):
                t = nb * k + 1 + s0
                s = (1 + s0) % nb
                wait_wb(t - 1, (s + nb - 1) % nb)
                g = t + nb - 1

                @pl.when(g <= n_tot - 1)
                def _():
                    start_gather(g, (s + nb - 1) % nb)

                wait_gather(t, s)
                start_wb(t, s)
            return carry

        lax.fori_loop(0, n_blocks, body, 0)
        for t in range(nb * n_blocks + 1, n_tot):
            step(t)
        wait_wb(n_tot - 1, (n_tot - 1) % nb)

    return gather_kernel


def kernel(x, table):
    V, D = table.shape
    B0, B1 = x.shape
    idx = jnp.transpose(
        jnp.reshape(jnp.transpose(x), (B1, B0 // 2, 2)), (0, 2, 1)
    ).astype(jnp.int32)
    NJ, NP = 8, 4
    chunk_p = 256
    out = _make_gather(V, D, B0, B1, NJ, NP, chunk_p)(idx, table)
    out4 = jnp.reshape(out, (B1, B0 // 2, 2, D))
    return jnp.reshape(jnp.transpose(out4, (1, 2, 0, 3)), (B0, B1, D))
